# baseline (speedup 1.0000x reference)
import functools

import jax
import jax.numpy as jnp
from jax import lax
from jax.experimental import pallas as pl
from jax.experimental.pallas import tpu as pltpu
from jax.experimental.pallas import tpu_sc as plsc

B = 16
L = 2048
DIM = 768
HALF = DIM // 2
NC = 2
NS = 16
LANES = 16
K = 32
G = HALF // LANES
HL = L // 2


def _sc_body(ids_hbm, mask_hbm, tab_hbm, out_hbm, idx_v, mask_v, rows_v, acc_v,
             cnt_v):
    c = lax.axis_index("c")
    s = lax.axis_index("s")
    w = s * NC + c
    i = w // 2
    h = w % 2

    pltpu.sync_copy(ids_hbm.at[i], idx_v.at[pl.ds(0, L)])
    pltpu.sync_copy(mask_hbm.at[i], mask_v)

    def count_body(j, cnt):
        m = mask_v[pl.ds(j * LANES, LANES)]
        return cnt + jnp.where(m == 1, 1, 0).astype(jnp.int32)

    cnt = lax.fori_loop(0, L // LANES, count_body,
                        jnp.zeros((LANES,), jnp.int32), unroll=4)
    n = cnt[0]
    for t in range(1, LANES):
        n = n + cnt[t]

    def xform_body(j, _):
        v = idx_v[pl.ds(j * LANES, LANES)]
        idx_v[pl.ds(j * LANES, LANES)] = v * 2 + h
        return 0

    lax.fori_loop(0, L // LANES, xform_body, 0, unroll=4)
    for t in range(K // LANES):
        idx_v[pl.ds(L + t * LANES, LANES)] = jnp.zeros((LANES,), jnp.int32)

    zeros = [jnp.zeros((LANES,), jnp.float32) for _ in range(G)]

    def chunk_body(j, acc):
        pltpu.sync_copy(tab_hbm.at[idx_v.at[pl.ds(j * K, K)]], rows_v)
        acc = list(acc)
        for r in range(K):
            for g in range(G):
                acc[g] = acc[g] + rows_v[r, pl.ds(g * LANES, LANES)]
        return tuple(acc)

    nfull = n // K
    acc = lax.fori_loop(0, nfull, chunk_body, tuple(zeros))

    base = nfull * K
    pltpu.sync_copy(tab_hbm.at[idx_v.at[pl.ds(base, K)]], rows_v)
    acc = list(acc)
    for r in range(K):
        wgt = jnp.where(base + r < n, 1.0, 0.0).astype(jnp.float32)
        for g in range(G):
            acc[g] = acc[g] + rows_v[r, pl.ds(g * LANES, LANES)] * wgt

    nf = n.astype(jnp.float32)
    for g in range(G):
        acc_v[pl.ds(g * LANES, LANES)] = acc[g] / nf
    pltpu.sync_copy(acc_v, out_hbm.at[w])


@jax.jit
def _sc_call(ids, mask, tab):
    mesh = plsc.VectorSubcoreMesh(core_axis_name="c", subcore_axis_name="s",
                                  num_cores=NC, num_subcores=NS)
    fn = pl.kernel(
        _sc_body,
        out_type=jax.ShapeDtypeStruct((NC * NS, HALF), jnp.float32),
        mesh=mesh,
        scratch_types=[
            pltpu.VMEM((L + K,), jnp.int32),
            pltpu.VMEM((L,), jnp.int32),
            pltpu.VMEM((K, HALF), jnp.float32),
            pltpu.VMEM((HALF,), jnp.float32),
            pltpu.VMEM((LANES,), jnp.int32),
        ],
    )
    return fn(ids, mask, tab)


def kernel(tag_input_ids, tag_attention_mask, table):
    ids = tag_input_ids.astype(jnp.int32)
    mask = tag_attention_mask.astype(jnp.int32)
    tab = table.reshape(2 * table.shape[0], HALF)
    out2 = _sc_call(ids, mask, tab)
    return out2.reshape(B, DIM)


# speedup vs baseline: 1.1060x; 1.1060x over previous
import jax
import jax.numpy as jnp
from jax import lax
from jax.experimental import pallas as pl
from jax.experimental.pallas import tpu as pltpu
from jax.experimental.pallas import tpu_sc as plsc

B = 16
L = 2048
DIM = 768
HALF = DIM // 2
NC = 2
NS = 16
LANES = 16
K = 32
G = HALF // LANES


def _sc_body(ids_hbm, mask_hbm, tab_hbm, out_hbm, idx_v, mask_v, rows_v,
             acc_v, sem):
    c = lax.axis_index("c")
    s = lax.axis_index("s")
    w = s * NC + c
    i = w // 2
    h = w % 2

    ids_cp = pltpu.make_async_copy(ids_hbm.at[i], idx_v.at[pl.ds(0, L)], sem)
    ids_cp.start()
    pltpu.sync_copy(mask_hbm.at[i], mask_v)
    ids_cp.wait()

    def prep_body(t, cnt):
        m = mask_v[pl.ds(t * LANES, LANES)]
        v = idx_v[pl.ds(t * LANES, LANES)]
        idx_v[pl.ds(t * LANES, LANES)] = v * 2 + h
        return cnt + jnp.where(m == 1, 1, 0).astype(jnp.int32)

    cnt = lax.fori_loop(0, L // LANES, prep_body,
                        jnp.zeros((LANES,), jnp.int32), unroll=4)
    n = cnt[0]
    for t in range(1, LANES):
        n = n + cnt[t]
    for t in range(K // LANES):
        idx_v[pl.ds(L + t * LANES, LANES)] = jnp.zeros((LANES,), jnp.int32)

    T = (n + K - 1) // K

    def chunk_copy(j):
        slot = lax.rem(j, 2)
        return pltpu.make_async_copy(
            tab_hbm.at[idx_v.at[pl.ds(j * K, K)]], rows_v.at[slot], sem)

    @pl.when(T > 0)
    def _():
        chunk_copy(0).start()

    zeros = [jnp.zeros((LANES,), jnp.float32) for _ in range(G)]

    def loop_body(j, acc):
        @pl.when(j + 1 < T)
        def _():
            chunk_copy(j + 1).start()
        chunk_copy(j).wait()
        slot = lax.rem(j, 2)
        base = j * K
        acc = list(acc)
        for r in range(K):
            wgt = jnp.where(base + r < n, 1.0, 0.0).astype(jnp.float32)
            for g in range(G):
                acc[g] = acc[g] + rows_v[slot, r, pl.ds(g * LANES, LANES)] * wgt
        return tuple(acc)

    acc = lax.fori_loop(0, T, loop_body, tuple(zeros))

    nf = n.astype(jnp.float32)
    for g in range(G):
        acc_v[pl.ds(g * LANES, LANES)] = acc[g] / nf
    pltpu.sync_copy(acc_v, out_hbm.at[w])


@jax.jit
def _sc_call(ids, mask, tab):
    mesh = plsc.VectorSubcoreMesh(core_axis_name="c", subcore_axis_name="s",
                                  num_cores=NC, num_subcores=NS)
    fn = pl.kernel(
        _sc_body,
        out_type=jax.ShapeDtypeStruct((NC * NS, HALF), jnp.float32),
        mesh=mesh,
        scratch_types=[
            pltpu.VMEM((L + K,), jnp.int32),
            pltpu.VMEM((L,), jnp.int32),
            pltpu.VMEM((2, K, HALF), jnp.float32),
            pltpu.VMEM((HALF,), jnp.float32),
            pltpu.SemaphoreType.DMA,
        ],
    )
    return fn(ids, mask, tab)


def kernel(tag_input_ids, tag_attention_mask, table):
    ids = tag_input_ids.astype(jnp.int32)
    mask = tag_attention_mask.astype(jnp.int32)
    tab = table.reshape(2 * table.shape[0], HALF)
    out2 = _sc_call(ids, mask, tab)
    return out2.reshape(B, DIM)


# speedup vs baseline: 1.3033x; 1.1784x over previous
import jax
import jax.numpy as jnp
from jax import lax
from jax.experimental import pallas as pl
from jax.experimental.pallas import tpu as pltpu
from jax.experimental.pallas import tpu_sc as plsc

B = 16
L = 2048
DIM = 768
HALF = DIM // 2
NC = 2
NS = 16
LANES = 16
K = 128
RB = 16
G = HALF // LANES


def _sc_body(ids_hbm, mask_hbm, tab_hbm, out_hbm, idx_v, mask_v, rows_v,
             acc_v, sem):
    c = lax.axis_index("c")
    s = lax.axis_index("s")
    w = s * NC + c
    i = w // 2
    h = w % 2

    ids_cp = pltpu.make_async_copy(ids_hbm.at[i], idx_v.at[pl.ds(0, L)], sem)
    ids_cp.start()
    pltpu.sync_copy(mask_hbm.at[i], mask_v)
    ids_cp.wait()

    def prep_body(t, cnt):
        m = mask_v[pl.ds(t * LANES, LANES)]
        v = idx_v[pl.ds(t * LANES, LANES)]
        idx_v[pl.ds(t * LANES, LANES)] = v * 2 + h
        return cnt + jnp.where(m == 1, 1, 0).astype(jnp.int32)

    cnt = lax.fori_loop(0, L // LANES, prep_body,
                        jnp.zeros((LANES,), jnp.int32), unroll=4)
    n = cnt[0]
    for t in range(1, LANES):
        n = n + cnt[t]
    for t in range(K // LANES):
        idx_v[pl.ds(L + t * LANES, LANES)] = jnp.zeros((LANES,), jnp.int32)

    T = (n + K - 1) // K

    def chunk_copy(j):
        slot = lax.rem(j, 2)
        return pltpu.make_async_copy(
            tab_hbm.at[idx_v.at[pl.ds(j * K, K)]], rows_v.at[slot], sem)

    @pl.when(T > 0)
    def _():
        chunk_copy(0).start()

    zeros = [jnp.zeros((LANES,), jnp.float32) for _ in range(G)]

    def loop_body(j, acc):
        @pl.when(j + 1 < T)
        def _():
            chunk_copy(j + 1).start()
        chunk_copy(j).wait()
        slot = lax.rem(j, 2)
        base = j * K

        def block_body(sub, acc):
            acc = list(acc)
            for r in range(RB):
                row = sub * RB + r
                wgt = jnp.where(base + row < n, 1.0, 0.0).astype(jnp.float32)
                for g in range(G):
                    acc[g] = acc[g] + (
                        rows_v[slot, row, pl.ds(g * LANES, LANES)] * wgt)
            return tuple(acc)

        return lax.fori_loop(0, K // RB, block_body, acc)

    acc = lax.fori_loop(0, T, loop_body, tuple(zeros))

    nf = n.astype(jnp.float32)
    for g in range(G):
        acc_v[pl.ds(g * LANES, LANES)] = acc[g] / nf
    pltpu.sync_copy(acc_v, out_hbm.at[w])


@jax.jit
def _sc_call(ids, mask, tab):
    mesh = plsc.VectorSubcoreMesh(core_axis_name="c", subcore_axis_name="s",
                                  num_cores=NC, num_subcores=NS)
    fn = pl.kernel(
        _sc_body,
        out_type=jax.ShapeDtypeStruct((NC * NS, HALF), jnp.float32),
        mesh=mesh,
        scratch_types=[
            pltpu.VMEM((L + K,), jnp.int32),
            pltpu.VMEM((L,), jnp.int32),
            pltpu.VMEM((2, K, HALF), jnp.float32),
            pltpu.VMEM((HALF,), jnp.float32),
            pltpu.SemaphoreType.DMA,
        ],
    )
    return fn(ids, mask, tab)


def kernel(tag_input_ids, tag_attention_mask, table):
    ids = tag_input_ids.astype(jnp.int32)
    mask = tag_attention_mask.astype(jnp.int32)
    tab = table.reshape(2 * table.shape[0], HALF)
    out2 = _sc_call(ids, mask, tab)
    return out2.reshape(B, DIM)
